# baseline (speedup 1.0000x reference)
import functools

import jax
import jax.numpy as jnp
from jax import lax
from jax.experimental import pallas as pl
from jax.experimental.pallas import tpu as pltpu
from jax.experimental.pallas import tpu_sc as plsc

_NW = 32
_W = 16
_NBUF = 6
_LEAD = 3


@functools.lru_cache(maxsize=None)
def _make_gather(B, V, D):
    b_per_w = B // _NW
    ch = b_per_w // _W
    mesh = plsc.VectorSubcoreMesh(core_axis_name="c", subcore_axis_name="s")

    scratch = [pltpu.VMEM((ch, _W), jnp.int32)]
    scratch += [pltpu.VMEM((_W, D), jnp.float32) for _ in range(_NBUF)]
    scratch += [pltpu.SemaphoreType.DMA for _ in range(2 * _NBUF)]

    @functools.partial(
        pl.kernel,
        mesh=mesh,
        out_type=jax.ShapeDtypeStruct((B, D), jnp.float32),
        scratch_types=scratch,
    )
    def k(ids_hbm, table_hbm, out_hbm, idx_v, *rest):
        bufs = list(rest[:_NBUF])
        gsem = list(rest[_NBUF:2 * _NBUF])
        ssem = list(rest[2 * _NBUF:])
        wid = lax.axis_index("s") * 2 + lax.axis_index("c")
        base = wid * b_per_w

        pltpu.sync_copy(ids_hbm.at[wid], idx_v)

        def g_copy(c):
            b = c % _NBUF
            return pltpu.make_async_copy(
                table_hbm.at[idx_v.at[c]], bufs[b], gsem[b])

        def s_copy(c):
            b = c % _NBUF
            return pltpu.make_async_copy(
                bufs[b], out_hbm.at[pl.ds(base + c * _W, _W)], ssem[b])

        for x in range(min(_LEAD, ch)):
            g_copy(x).start()
        for c in range(ch):
            x = c + _LEAD
            if x < ch:
                if x - _NBUF >= 0:
                    s_copy(x - _NBUF).wait()
                g_copy(x).start()
            g_copy(c).wait()
            s_copy(c).start()
        for c in range(max(0, ch - _NBUF), ch):
            s_copy(c).wait()

    return k


def kernel(token_ids, table):
    B0, B1 = token_ids.shape
    B = B0 * B1
    V, D = table.shape
    ids = token_ids.reshape(_NW, (B // _NW) // _W, _W)
    out = _make_gather(B, V, D)(ids, table)
    return out.reshape(B0, B1, D)


# speedup vs baseline: 1.0224x; 1.0224x over previous
import functools

import jax
import jax.numpy as jnp
from jax import lax
from jax.experimental import pallas as pl
from jax.experimental.pallas import tpu as pltpu
from jax.experimental.pallas import tpu_sc as plsc

_NW = 32
_W = 16
_NBUF = 4
_LEAD = 2


@functools.lru_cache(maxsize=None)
def _make_gather(B, V, D):
    b_per_w = B // _NW
    ch = b_per_w // _W
    assert ch % _NBUF == 0 and ch // _NBUF >= 2 and _LEAD < _NBUF
    mesh = plsc.VectorSubcoreMesh(core_axis_name="c", subcore_axis_name="s")

    scratch = [pltpu.VMEM((ch, _W), jnp.int32)]
    scratch += [pltpu.VMEM((_W, D), jnp.float32) for _ in range(_NBUF)]
    scratch += [pltpu.SemaphoreType.DMA for _ in range(2 * _NBUF)]

    @functools.partial(
        pl.kernel,
        mesh=mesh,
        out_type=jax.ShapeDtypeStruct((B, D), jnp.float32),
        scratch_types=scratch,
    )
    def k(ids_hbm, table_hbm, out_hbm, idx_v, *rest):
        bufs = list(rest[:_NBUF])
        gsem = list(rest[_NBUF:2 * _NBUF])
        ssem = list(rest[2 * _NBUF:])
        wid = lax.axis_index("s") * 2 + lax.axis_index("c")
        base = wid * b_per_w

        pltpu.sync_copy(ids_hbm.at[wid], idx_v)

        def g_copy(c, b):
            return pltpu.make_async_copy(
                table_hbm.at[idx_v.at[c]], bufs[b], gsem[b])

        def s_copy(c, b):
            return pltpu.make_async_copy(
                bufs[b], out_hbm.at[pl.ds(base + c * _W, _W)], ssem[b])

        for x in range(min(_LEAD, ch)):
            g_copy(x, x % _NBUF).start()
        for c in range(_NBUF):
            x = c + _LEAD
            if x < ch:
                if x - _NBUF >= 0:
                    s_copy(x - _NBUF, x % _NBUF).wait()
                g_copy(x, x % _NBUF).start()
            g_copy(c, c % _NBUF).wait()
            s_copy(c, c % _NBUF).start()

        def body(g, carry):
            c0 = g * _NBUF
            for b in range(_NBUF):
                c = c0 + b
                x = c + _LEAD
                bx = (b + _LEAD) % _NBUF
                s_copy(x - _NBUF, bx).wait()
                g_copy(x, bx).start()
                g_copy(c, b).wait()
                s_copy(c, b).start()
            return carry

        lax.fori_loop(1, ch // _NBUF - 1, body, 0)

        for c in range(ch - _NBUF, ch):
            x = c + _LEAD
            if x < ch:
                s_copy(x - _NBUF, x % _NBUF).wait()
                g_copy(x, x % _NBUF).start()
            g_copy(c, c % _NBUF).wait()
            s_copy(c, c % _NBUF).start()
        for c in range(ch - _NBUF, ch):
            s_copy(c, c % _NBUF).wait()

    return k


def kernel(token_ids, table):
    B0, B1 = token_ids.shape
    B = B0 * B1
    V, D = table.shape
    ids = token_ids.reshape(_NW, (B // _NW) // _W, _W)
    out = _make_gather(B, V, D)(ids, table)
    return out.reshape(B0, B1, D)
